# Initial kernel scaffold; baseline (speedup 1.0000x reference)
import jax
import jax.numpy as jnp
from jax.experimental import pallas as pl


def kernel(x):
    raise NotImplementedError("write your pallas kernel here")


# speedup vs baseline: 10.7373x; 10.7373x over previous
import functools

import jax
import jax.numpy as jnp
from jax import lax
from jax.experimental import pallas as pl

_B = 128
_N = 32768
_KTH = 32704
_TOPK = _N - _KTH + 1
_BLOCK_ROWS = 8


def _float_key(x):
    bits = lax.bitcast_convert_type(x, jnp.int32)
    return jnp.where(bits < 0, bits ^ jnp.int32(0x7FFFFFFF), bits)


def _key_float(k):
    bits = jnp.where(k < 0, k ^ jnp.int32(0x7FFFFFFF), k)
    return lax.bitcast_convert_type(bits, jnp.float32)


def _body(x_ref, y_ref, maxy_ref):
    xb = x_ref[...]
    m = jnp.max(xb, axis=1, keepdims=True)
    e = jnp.exp(xb - m)
    s = jnp.sum(e, axis=1, keepdims=True)

    key = _float_key(xb)

    cnt_nonneg = jnp.sum((key >= 0).astype(jnp.int32), axis=1, keepdims=True)
    t = jnp.where(cnt_nonneg >= _TOPK,
                  jnp.zeros_like(cnt_nonneg),
                  jnp.full_like(cnt_nonneg, jnp.int32(-0x80000000)))

    def step(i, t):
        bit = jnp.int32(1) << (jnp.int32(30) - i)
        cand = t | bit
        cnt = jnp.sum((key >= cand).astype(jnp.int32), axis=1, keepdims=True)
        return jnp.where(cnt >= _TOPK, cand, t)

    t = lax.fori_loop(0, 31, step, t, unroll=True)

    xk = _key_float(t)
    inv_s = 1.0 / s
    c = jnp.exp(xk - m) * inv_s
    sm = e * inv_s
    y = jnp.minimum(jnp.maximum(sm - c, 0.0) * 10.0, 1.0)
    y_ref[...] = y
    maxy_ref[...] = jnp.broadcast_to(jnp.max(y, axis=1, keepdims=True),
                                     maxy_ref.shape)


@jax.jit
def kernel(x):
    grid = _B // _BLOCK_ROWS
    y, maxy = pl.pallas_call(
        _body,
        grid=(grid,),
        in_specs=[pl.BlockSpec((_BLOCK_ROWS, _N), lambda i: (i, 0))],
        out_specs=[
            pl.BlockSpec((_BLOCK_ROWS, _N), lambda i: (i, 0)),
            pl.BlockSpec((_BLOCK_ROWS, 128), lambda i: (i, 0)),
        ],
        out_shape=[
            jax.ShapeDtypeStruct((_B, _N), jnp.float32),
            jax.ShapeDtypeStruct((_B, 128), jnp.float32),
        ],
    )(x)

    cond = jnp.max(maxy) == 0.0
    return lax.cond(cond, lambda: jax.nn.softmax(x, axis=1), lambda: y)


# speedup vs baseline: 16.7022x; 1.5555x over previous
import functools

import jax
import jax.numpy as jnp
from jax import lax
from jax.experimental import pallas as pl
from jax.experimental.pallas import tpu as pltpu
from jax.experimental.pallas import tpu_sc as plsc

_B = 128
_N = 32768
_TOPK = 65
_L = 16
_NV = _N // _L
_NACC = 8
_NPAIR = _NV // 2

_NC = 2
_NS = 16
_NW = _NC * _NS
_RPW = _B // _NW

_BIS_A = 36
_BIS_B = 36
_UNR_B = 8

_MYW = 1024


def _bfly_max(v, idx):
    for sh in (8, 4, 2, 1):
        v = jnp.maximum(v, v[idx ^ sh])
    return v


def _bfly_min(v, idx):
    for sh in (8, 4, 2, 1):
        v = jnp.minimum(v, v[idx ^ sh])
    return v


def _bfly_sum(v, idx):
    for sh in (8, 4, 2, 1):
        v = v + v[idx ^ sh]
    return v


def _sc_body(x_hbm, y_hbm, maxy_hbm, xv, cand, sem):
    idx = lax.iota(jnp.int32, _L)
    wid = lax.axis_index("s") * _NC + lax.axis_index("c")
    one_i = jnp.full((_L,), 1, jnp.int32)
    zero_i = jnp.full((_L,), 0, jnp.int32)
    topk_v = jnp.full((_L,), _TOPK, jnp.int32)
    neginf = jnp.full((_L,), -jnp.inf, jnp.float32)

    def row_body(j, _):
        r = wid * _RPW + j
        pltpu.sync_copy(x_hbm.at[r], xv)

        def p1(i, accs):
            base = i * _L * _L
            out = []
            for k in range(_NACC):
                v0 = xv[pl.ds(base + k * _L, _L)]
                v1 = xv[pl.ds(base + (k + _NACC) * _L, _L)]
                out.append(jnp.maximum(accs[k], jnp.maximum(v0, v1)))
            return tuple(out)

        accs = lax.fori_loop(0, _NV // _L, p1, tuple([neginf] * _NACC))

        amax = accs[0]
        amin = accs[0]
        for k in range(1, _NACC):
            amax = jnp.maximum(amax, accs[k])
            amin = jnp.minimum(amin, accs[k])
        m_v = _bfly_max(amax, idx)
        lo0 = _bfly_min(amin, idx)
        hi0 = m_v + 1.0

        def bis_a(i, lohi):
            lo, hi = lohi
            mid = lo * 0.5 + hi * 0.5
            cnt = zero_i
            for k in range(_NACC):
                cnt = cnt + jnp.where(accs[k] >= mid, one_i, zero_i)
            tot = _bfly_sum(cnt, idx)
            ge = tot >= topk_v
            return (jnp.where(ge, mid, lo), jnp.where(ge, hi, mid))

        t_v, _hi = lax.fori_loop(0, _BIS_A, bis_a, (lo0, hi0))

        def p2(i, carry):
            off, acc = carry
            for u in range(2):
                p = i * 2 + u
                v0 = xv[pl.ds((2 * p) * _L, _L)]
                v1 = xv[pl.ds((2 * p + 1) * _L, _L)]
                acc = acc + (jnp.exp(v0 - m_v) + jnp.exp(v1 - m_v))
                pm = _bfly_max(jnp.maximum(v0, v1), idx)
                cand[pl.ds(off, _L)] = v0
                cand[pl.ds(off + _L, _L)] = v1
                adv = jnp.where(pm >= t_v, jnp.full((_L,), 2 * _L, jnp.int32),
                                zero_i)
                off = off + adv[0]
            return off, acc

        off, acc = lax.fori_loop(
            0, _NPAIR // 2, p2,
            (jnp.int32(0), jnp.full((_L,), 0.0, jnp.float32)))

        for k in range(_UNR_B):
            cand[pl.ds(off + k * _L, _L)] = neginf

        ntrip = (off // _L + jnp.int32(_UNR_B - 1)) // _UNR_B

        def bis_b(i, lohi):
            lo, hi = lohi
            mid = lo * 0.5 + hi * 0.5

            def count8(jj, cnt):
                for k in range(_UNR_B):
                    c = cand[pl.ds((jj * _UNR_B + k) * _L, _L)]
                    cnt = cnt + jnp.where(c >= mid, one_i, zero_i)
                return cnt

            cnt = lax.fori_loop(0, ntrip, count8, zero_i)
            tot = _bfly_sum(cnt, idx)
            ge = tot >= topk_v
            return (jnp.where(ge, mid, lo), jnp.where(ge, hi, mid))

        xk_v, _hi2 = lax.fori_loop(0, _BIS_B, bis_b, (t_v, hi0))

        s_v = _bfly_sum(acc, idx)
        inv_s = 1.0 / s_v
        c_v = jnp.exp(xk_v - m_v) * inv_s

        def p3(i, my):
            for k in range(_L):
                dsl = pl.ds((i * _L + k) * _L, _L)
                sm = jnp.exp(xv[dsl] - m_v) * inv_s
                yv = jnp.minimum(jnp.maximum(sm - c_v, 0.0) * 10.0, 1.0)
                xv[dsl] = yv
                my = jnp.maximum(my, yv)
            return my

        my = lax.fori_loop(0, _NV // _L, p3,
                           jnp.full((_L,), 0.0, jnp.float32))

        pltpu.sync_copy(xv, y_hbm.at[r])

        for k in range(_MYW // _L):
            cand[pl.ds(k * _L, _L)] = my
        pltpu.sync_copy(cand.at[pl.ds(0, _MYW)], maxy_hbm.at[r])
        return 0

    lax.fori_loop(0, _RPW, row_body, 0)


_sc_call = functools.partial(
    pl.kernel,
    mesh=plsc.VectorSubcoreMesh(core_axis_name="c", subcore_axis_name="s"),
    out_type=[
        jax.ShapeDtypeStruct((_B, _N), jnp.float32),
        jax.ShapeDtypeStruct((_B, _MYW), jnp.float32),
    ],
    scratch_types=[
        pltpu.VMEM((_N,), jnp.float32),
        pltpu.VMEM((_N + _UNR_B * _L,), jnp.float32),
        pltpu.SemaphoreType.DMA,
    ],
)(_sc_body)


@jax.jit
def kernel(x):
    y, maxy = _sc_call(x)
    cond = jnp.max(maxy) == 0.0
    return lax.cond(cond, lambda: jax.nn.softmax(x, axis=1), lambda: y)


# speedup vs baseline: 20.2746x; 1.2139x over previous
import functools

import jax
import jax.numpy as jnp
from jax import lax
from jax.experimental import pallas as pl
from jax.experimental.pallas import tpu as pltpu
from jax.experimental.pallas import tpu_sc as plsc

_B = 128
_N = 32768
_TOPK = 65
_L = 16
_NV = _N // _L
_NACC = 8
_NU = _NV // 4

_NC = 2
_NS = 16
_NW = _NC * _NS
_RPW = _B // _NW

_BIS_A = 22
_BIS_B = 20
_UNR_B = 8
_UNR_R = 2

_MYW = 1024


def _bfly_max(v, idx):
    for sh in (8, 4, 2, 1):
        v = jnp.maximum(v, v[idx ^ sh])
    return v


def _bfly_min(v, idx):
    for sh in (8, 4, 2, 1):
        v = jnp.minimum(v, v[idx ^ sh])
    return v


def _bfly_sum(v, idx):
    for sh in (8, 4, 2, 1):
        v = v + v[idx ^ sh]
    return v


def _sc_body(x_hbm, y_hbm, maxy_hbm, xv, cand, flagv, sem):
    idx = lax.iota(jnp.int32, _L)
    wid = lax.axis_index("s") * _NC + lax.axis_index("c")
    one_i = jnp.full((_L,), 1, jnp.int32)
    zero_i = jnp.full((_L,), 0, jnp.int32)
    topk_v = jnp.full((_L,), _TOPK, jnp.int32)
    neginf = jnp.full((_L,), -jnp.inf, jnp.float32)

    def row_body(j, _):
        r = wid * _RPW + j
        pltpu.sync_copy(x_hbm.at[r], xv)

        def p1(i, accs):
            base = i * _L * _L
            out = []
            for k in range(_NACC):
                v0 = xv[pl.ds(base + k * _L, _L)]
                v1 = xv[pl.ds(base + (k + _NACC) * _L, _L)]
                out.append(jnp.maximum(accs[k], jnp.maximum(v0, v1)))
            return tuple(out)

        accs = lax.fori_loop(0, _NV // _L, p1, tuple([neginf] * _NACC))

        amax = accs[0]
        amin = accs[0]
        for k in range(1, _NACC):
            amax = jnp.maximum(amax, accs[k])
            amin = jnp.minimum(amin, accs[k])
        m_v = _bfly_max(amax, idx)
        lo0 = _bfly_min(amin, idx)
        hi0 = m_v + 1.0

        def bis_a(i, lohi):
            lo, hi = lohi
            mid = lo * 0.5 + hi * 0.5
            cnt = zero_i
            for k in range(_NACC):
                cnt = cnt + jnp.where(accs[k] >= mid, one_i, zero_i)
            tot = _bfly_sum(cnt, idx)
            ge = tot >= topk_v
            return (jnp.where(ge, mid, lo), jnp.where(ge, hi, mid))

        t_v, _hi = lax.fori_loop(0, _BIS_A, bis_a, (lo0, hi0))

        def p2(i, carry):
            off, acc = carry
            b = i * 4 * _L
            v0 = xv[pl.ds(b, _L)]
            v1 = xv[pl.ds(b + _L, _L)]
            v2 = xv[pl.ds(b + 2 * _L, _L)]
            v3 = xv[pl.ds(b + 3 * _L, _L)]
            e01 = jnp.exp(v0 - m_v) + jnp.exp(v1 - m_v)
            e23 = jnp.exp(v2 - m_v) + jnp.exp(v3 - m_v)
            acc = acc + (e01 + e23)
            pm = jnp.maximum(jnp.maximum(v0, v1), jnp.maximum(v2, v3))
            pm = _bfly_max(pm, idx)
            cand[pl.ds(off, _L)] = v0
            cand[pl.ds(off + _L, _L)] = v1
            cand[pl.ds(off + 2 * _L, _L)] = v2
            cand[pl.ds(off + 3 * _L, _L)] = v3
            adv = jnp.where(pm >= t_v, jnp.full((_L,), 4 * _L, jnp.int32),
                            zero_i)
            return off + adv[0], acc

        off, acc = lax.fori_loop(
            0, _NU, p2, (jnp.int32(0), jnp.full((_L,), 0.0, jnp.float32)))

        for k in range(_UNR_R):
            cand[pl.ds(off + k * _L, _L)] = neginf

        ntr_r = (off // _L + jnp.int32(_UNR_R - 1)) // _UNR_R

        def recomp(i, woff):
            for k in range(_UNR_R):
                v = cand[pl.ds((i * _UNR_R + k) * _L, _L)]
                pm = _bfly_max(v, idx)
                cand[pl.ds(woff, _L)] = v
                adv = jnp.where(pm >= t_v, jnp.full((_L,), _L, jnp.int32),
                                zero_i)
                woff = woff + adv[0]
            return woff

        woff = lax.fori_loop(0, ntr_r, recomp, jnp.int32(0))

        for k in range(_UNR_B):
            cand[pl.ds(woff + k * _L, _L)] = neginf

        ntrip = (woff // _L + jnp.int32(_UNR_B - 1)) // _UNR_B

        def bis_b(i, lohi):
            lo, hi = lohi
            mid = lo * 0.5 + hi * 0.5

            def count8(jj, cnt):
                for k in range(_UNR_B):
                    c = cand[pl.ds((jj * _UNR_B + k) * _L, _L)]
                    cnt = cnt + jnp.where(c >= mid, one_i, zero_i)
                return cnt

            cnt = lax.fori_loop(0, ntrip, count8, zero_i)
            tot = _bfly_sum(cnt, idx)
            ge = tot >= topk_v
            return (jnp.where(ge, mid, lo), jnp.where(ge, hi, mid))

        xk_v, _hi2 = lax.fori_loop(0, _BIS_B, bis_b, (t_v, hi0))

        s_v = _bfly_sum(acc, idx)
        r10 = 10.0 / s_v
        c10 = jnp.exp(xk_v - m_v) * r10

        def p3(i, _c):
            for k in range(_L):
                dsl = pl.ds((i * _L + k) * _L, _L)
                sc = jnp.exp(xv[dsl] - m_v) * r10
                xv[dsl] = jnp.minimum(jnp.maximum(sc - c10, 0.0), 1.0)
            return 0

        lax.fori_loop(0, _NV // _L, p3, 0)

        pltpu.sync_copy(xv, y_hbm.at[r])

        flagv[...] = jnp.where(r10 <= c10, jnp.full((_L,), 0.0, jnp.float32),
                               jnp.full((_L,), 1.0, jnp.float32))
        pltpu.sync_copy(flagv, maxy_hbm.at[r, pl.ds(0, _L)])
        return 0

    lax.fori_loop(0, _RPW, row_body, 0)


_sc_call = functools.partial(
    pl.kernel,
    mesh=plsc.VectorSubcoreMesh(core_axis_name="c", subcore_axis_name="s"),
    out_type=[
        jax.ShapeDtypeStruct((_B, _N), jnp.float32),
        jax.ShapeDtypeStruct((_B, _MYW), jnp.float32),
    ],
    scratch_types=[
        pltpu.VMEM((_N,), jnp.float32),
        pltpu.VMEM((_N + _UNR_B * _L,), jnp.float32),
        pltpu.VMEM((_L,), jnp.float32),
        pltpu.SemaphoreType.DMA,
    ],
)(_sc_body)


@jax.jit
def kernel(x):
    y, flags = _sc_call(x)
    cond = jnp.max(flags[:, :_L]) == 0.0
    return lax.cond(cond, lambda: jax.nn.softmax(x, axis=1), lambda: y)


# speedup vs baseline: 20.9466x; 1.0331x over previous
import functools

import jax
import jax.numpy as jnp
from jax import lax
from jax.experimental import pallas as pl
from jax.experimental.pallas import tpu as pltpu
from jax.experimental.pallas import tpu_sc as plsc

_B = 128
_N = 32768
_TOPK = 65
_L = 16
_NV = _N // _L
_NACC = 8
_NU = _NV // 4

_NC = 2
_NS = 16
_NW = _NC * _NS
_RPW = _B // _NW

_BIS_A = 22
_BIS_B = 20
_UNR_B = 8
_UNR_R = 2

_MYW = 1024


def _bfly_max(v, idx):
    for sh in (8, 4, 2, 1):
        v = jnp.maximum(v, v[idx ^ sh])
    return v


def _bfly_min(v, idx):
    for sh in (8, 4, 2, 1):
        v = jnp.minimum(v, v[idx ^ sh])
    return v


def _bfly_sum(v, idx):
    for sh in (8, 4, 2, 1):
        v = v + v[idx ^ sh]
    return v


def _sc_body(x_hbm, y_hbm, maxy_hbm, xv0, xv1, cand, flagv,
             si0, si1, so0, so1):
    idx = lax.iota(jnp.int32, _L)
    wid = lax.axis_index("s") * _NC + lax.axis_index("c")
    one_i = jnp.full((_L,), 1, jnp.int32)
    zero_i = jnp.full((_L,), 0, jnp.int32)
    topk_v = jnp.full((_L,), _TOPK, jnp.int32)
    neginf = jnp.full((_L,), -jnp.inf, jnp.float32)
    bufs = (xv0, xv1)
    isems = (si0, si1)
    osems = (so0, so1)
    r0 = wid * _RPW

    def row_compute(r, xv):
        def p1(i, accs):
            base = i * _L * _L
            out = []
            for k in range(_NACC):
                v0 = xv[pl.ds(base + k * _L, _L)]
                v1 = xv[pl.ds(base + (k + _NACC) * _L, _L)]
                out.append(jnp.maximum(accs[k], jnp.maximum(v0, v1)))
            return tuple(out)

        accs = lax.fori_loop(0, _NV // _L, p1, tuple([neginf] * _NACC))

        amax = accs[0]
        amin = accs[0]
        for k in range(1, _NACC):
            amax = jnp.maximum(amax, accs[k])
            amin = jnp.minimum(amin, accs[k])
        m_v = _bfly_max(amax, idx)
        lo0 = _bfly_min(amin, idx)
        hi0 = m_v + 1.0

        def bis_a(i, lohi):
            lo, hi = lohi
            mid = lo * 0.5 + hi * 0.5
            cnt = zero_i
            for k in range(_NACC):
                cnt = cnt + jnp.where(accs[k] >= mid, one_i, zero_i)
            tot = _bfly_sum(cnt, idx)
            ge = tot >= topk_v
            return (jnp.where(ge, mid, lo), jnp.where(ge, hi, mid))

        t_v, _hi = lax.fori_loop(0, _BIS_A, bis_a, (lo0, hi0))

        def p2(i, carry):
            off, acc = carry
            for u in range(2):
                b = (i * 2 + u) * 4 * _L
                v0 = xv[pl.ds(b, _L)]
                v1 = xv[pl.ds(b + _L, _L)]
                v2 = xv[pl.ds(b + 2 * _L, _L)]
                v3 = xv[pl.ds(b + 3 * _L, _L)]
                e01 = jnp.exp(v0 - m_v) + jnp.exp(v1 - m_v)
                e23 = jnp.exp(v2 - m_v) + jnp.exp(v3 - m_v)
                acc = acc + (e01 + e23)
                pm = jnp.maximum(jnp.maximum(v0, v1), jnp.maximum(v2, v3))
                pm = _bfly_max(pm, idx)
                cand[pl.ds(off, _L)] = v0
                cand[pl.ds(off + _L, _L)] = v1
                cand[pl.ds(off + 2 * _L, _L)] = v2
                cand[pl.ds(off + 3 * _L, _L)] = v3
                adv = jnp.where(pm >= t_v,
                                jnp.full((_L,), 4 * _L, jnp.int32), zero_i)
                off = off + adv[0]
            return off, acc

        off, acc = lax.fori_loop(
            0, _NU // 2, p2, (jnp.int32(0), jnp.full((_L,), 0.0, jnp.float32)))

        for k in range(_UNR_R):
            cand[pl.ds(off + k * _L, _L)] = neginf

        ntr_r = (off // _L + jnp.int32(_UNR_R - 1)) // _UNR_R

        def recomp(i, woff):
            for k in range(_UNR_R):
                v = cand[pl.ds((i * _UNR_R + k) * _L, _L)]
                pm = _bfly_max(v, idx)
                cand[pl.ds(woff, _L)] = v
                adv = jnp.where(pm >= t_v, jnp.full((_L,), _L, jnp.int32),
                                zero_i)
                woff = woff + adv[0]
            return woff

        woff = lax.fori_loop(0, ntr_r, recomp, jnp.int32(0))

        for k in range(_UNR_B):
            cand[pl.ds(woff + k * _L, _L)] = neginf

        ntrip = (woff // _L + jnp.int32(_UNR_B - 1)) // _UNR_B

        def bis_b(i, lohi):
            lo, hi = lohi
            mid = lo * 0.5 + hi * 0.5

            def count8(jj, cnt):
                for k in range(_UNR_B):
                    c = cand[pl.ds((jj * _UNR_B + k) * _L, _L)]
                    cnt = cnt + jnp.where(c >= mid, one_i, zero_i)
                return cnt

            cnt = lax.fori_loop(0, ntrip, count8, zero_i)
            tot = _bfly_sum(cnt, idx)
            ge = tot >= topk_v
            return (jnp.where(ge, mid, lo), jnp.where(ge, hi, mid))

        xk_v, _hi2 = lax.fori_loop(0, _BIS_B, bis_b, (t_v, hi0))

        s_v = _bfly_sum(acc, idx)
        r10 = 10.0 / s_v
        c10 = jnp.exp(xk_v - m_v) * r10

        def p3(i, _c):
            for k in range(_L):
                dsl = pl.ds((i * _L + k) * _L, _L)
                sc = jnp.exp(xv[dsl] - m_v) * r10
                xv[dsl] = jnp.minimum(jnp.maximum(sc - c10, 0.0), 1.0)
            return 0

        lax.fori_loop(0, _NV // _L, p3, 0)

        flagv[...] = jnp.where(r10 <= c10, jnp.full((_L,), 0.0, jnp.float32),
                               jnp.full((_L,), 1.0, jnp.float32))
        pltpu.sync_copy(flagv, maxy_hbm.at[r, pl.ds(0, _L)])

    in_h = {}
    out_h = {}
    in_h[0] = pltpu.async_copy(x_hbm.at[r0], bufs[0], isems[0])
    for j in range(_RPW):
        b = j & 1
        if j + 1 < _RPW:
            if j - 1 >= 0:
                out_h[j - 1].wait()
            in_h[j + 1] = pltpu.async_copy(
                x_hbm.at[r0 + j + 1], bufs[(j + 1) & 1], isems[(j + 1) & 1])
        in_h[j].wait()
        row_compute(r0 + j, bufs[b])
        out_h[j] = pltpu.async_copy(bufs[b], y_hbm.at[r0 + j], osems[b])
    out_h[_RPW - 2].wait()
    out_h[_RPW - 1].wait()


_sc_call = functools.partial(
    pl.kernel,
    mesh=plsc.VectorSubcoreMesh(core_axis_name="c", subcore_axis_name="s"),
    out_type=[
        jax.ShapeDtypeStruct((_B, _N), jnp.float32),
        jax.ShapeDtypeStruct((_B, _MYW), jnp.float32),
    ],
    scratch_types=[
        pltpu.VMEM((_N,), jnp.float32),
        pltpu.VMEM((_N,), jnp.float32),
        pltpu.VMEM((_N + _UNR_B * _L,), jnp.float32),
        pltpu.VMEM((_L,), jnp.float32),
        pltpu.SemaphoreType.DMA,
        pltpu.SemaphoreType.DMA,
        pltpu.SemaphoreType.DMA,
        pltpu.SemaphoreType.DMA,
    ],
)(_sc_body)


@jax.jit
def kernel(x):
    y, flags = _sc_call(x)
    cond = jnp.max(flags[:, :_L]) == 0.0
    return lax.cond(cond, lambda: jax.nn.softmax(x, axis=1), lambda: y)
